# Initial kernel scaffold; baseline (speedup 1.0000x reference)
import jax
import jax.numpy as jnp
from jax.experimental import pallas as pl


def kernel(X, state, W, Win, bias, Wout, sr, adaptive_lr, temperature, w_h, w_o, w_d, win_h, win_o, win_d):
    raise NotImplementedError("write your pallas kernel here")


# speedup vs baseline: 592.9008x; 592.9008x over previous
import jax
import jax.numpy as jnp
from jax.experimental import pallas as pl


def _est_body(xall_ref, x_ref, st_ref, w_ref, win_ref, b_ref, wout_ref,
              sr_ref, alr_ref, temp_ref, ns_ref, out_ref):
    u = pl.program_id(0)
    nu = pl.num_programs(0)
    temp = temp_ref[0, 0]

    x_all = xall_ref[...]
    alr = alr_ref[...][:, :, 0]
    logits = jnp.sum(x_all * alr[:, None, :], axis=-1) / temp
    m = jnp.max(logits, axis=0)
    e = jnp.exp(logits - m[None, :])
    denom = jnp.sum(e, axis=0)
    onehot = (jax.lax.broadcasted_iota(jnp.int32, (nu, 1), 0) == u
              ).astype(jnp.float32)
    lr_u = (jnp.sum(e * onehot, axis=0) / denom)[:, None]
    sr_u = jnp.sum(sr_ref[...][:, :, 0] * onehot)

    x_u = x_ref[0]
    st_u = st_ref[0]
    feed = jnp.dot(x_u, win_ref[0], preferred_element_type=jnp.float32)
    echo = jnp.dot(st_u * sr_u, w_ref[0], preferred_element_type=jnp.float32)
    act = jnp.tanh(feed + echo + b_ref[0, 0, :][None, :])
    ns = (1.0 - lr_u) * st_u + lr_u * act
    ns_ref[...] = ns[None, :, :]
    out_ref[...] = jnp.dot(ns, wout_ref[0],
                           preferred_element_type=jnp.float32)[None, :, :]


def kernel(X, state, W, Win, bias, Wout, sr, adaptive_lr, temperature,
           w_h, w_o, w_d, win_h, win_o, win_d):
    B, U, D = X.shape
    N = state.shape[2]
    O = Wout.shape[2]
    Xt = X.transpose(1, 0, 2)
    stt = state.transpose(1, 0, 2)
    temp2 = temperature.reshape(1, 1)
    ns, out = pl.pallas_call(
        _est_body,
        grid=(U,),
        in_specs=[
            pl.BlockSpec((U, B, D), lambda u: (0, 0, 0)),
            pl.BlockSpec((1, B, D), lambda u: (u, 0, 0)),
            pl.BlockSpec((1, B, N), lambda u: (u, 0, 0)),
            pl.BlockSpec((1, N, N), lambda u: (u, 0, 0)),
            pl.BlockSpec((1, D, N), lambda u: (u, 0, 0)),
            pl.BlockSpec((1, 1, N), lambda u: (u, 0, 0)),
            pl.BlockSpec((1, N, O), lambda u: (u, 0, 0)),
            pl.BlockSpec((U, 1, 1), lambda u: (0, 0, 0)),
            pl.BlockSpec((U, D, 1), lambda u: (0, 0, 0)),
            pl.BlockSpec((1, 1), lambda u: (0, 0)),
        ],
        out_specs=[
            pl.BlockSpec((1, B, N), lambda u: (u, 0, 0)),
            pl.BlockSpec((1, B, O), lambda u: (u, 0, 0)),
        ],
        out_shape=[
            jax.ShapeDtypeStruct((U, B, N), jnp.float32),
            jax.ShapeDtypeStruct((U, B, O), jnp.float32),
        ],
    )(Xt, Xt, stt, W, Win, bias, Wout, sr, adaptive_lr, temp2)
    return ns.transpose(1, 0, 2), out.transpose(1, 0, 2)
